# Initial kernel scaffold; baseline (speedup 1.0000x reference)
import jax
import jax.numpy as jnp
from jax.experimental import pallas as pl


def kernel(stock_reps, market_reps, params):
    raise NotImplementedError("write your pallas kernel here")


# speedup vs baseline: 2.1492x; 2.1492x over previous
import jax
import jax.numpy as jnp
from jax.experimental import pallas as pl

N_STOCKS = 2048
N_MARKET = 32
HIDDEN = 256
N_CLUSTERS = 8
N_SUBCLUSTERS = 4
N_HEADS = 4
DH = HIDDEN // N_HEADS
CSIZE = N_STOCKS // N_CLUSTERS
ISIZE = CSIZE // N_SUBCLUSTERS
QB = 256
NEG = -1000000000.0
_HI = jax.lax.Precision.HIGHEST


def _norm_rows(v, eps=1e-12):
    n = jnp.sqrt(jnp.sum(v * v, axis=-1, keepdims=True))
    return v / jnp.maximum(n, eps)


def _rank_kernel(sims_ref, cl_ref, idxs_ref):
    sims_r = sims_ref[...]
    sims_c = jnp.transpose(sims_r)
    idx_c = jax.lax.broadcasted_iota(jnp.int32, (N_STOCKS, 1), 0)
    idx_r = jax.lax.broadcasted_iota(jnp.int32, (1, N_STOCKS), 1)

    lt = (sims_r < sims_c) | ((sims_r == sims_c) & (idx_r < idx_c))
    rank = jnp.sum(lt.astype(jnp.float32), axis=1, keepdims=True)
    cl_c = (N_CLUSTERS - 1
            - (rank * (1.0 / CSIZE)).astype(jnp.int32)).astype(jnp.int32)
    cl_r = jnp.transpose(cl_c)

    same = (cl_r == cl_c)
    pos = jnp.sum((same & (idx_r < idx_c)).astype(jnp.float32), axis=1,
                  keepdims=True)
    piota = jax.lax.broadcasted_iota(jnp.int32, (1, CSIZE), 1)
    pos_i = pos.astype(jnp.int32)
    jf_r = idx_r.astype(jnp.float32)

    cl_ref[...] = cl_r
    for c in range(N_CLUSTERS):
        memb_c = (cl_c == c)
        ph = ((pos_i == piota) & memb_c).astype(jnp.float32)
        row = jax.lax.dot_general(jf_r, ph, (((1,), (0,)), ((), ())),
                                  preferred_element_type=jnp.float32,
                                  precision=_HI)
        idxs_ref[c, :] = row.astype(jnp.int32)[0]


def _interval_kernel(s_ref, idxs_ref, iv_ref):
    jot = jax.lax.broadcasted_iota(jnp.int32, (1, N_STOCKS), 1)
    q_c = jax.lax.broadcasted_iota(jnp.int32, (CSIZE, 1), 0)
    p_r = jax.lax.broadcasted_iota(jnp.int32, (1, CSIZE), 1)
    acc = jnp.zeros((1, N_STOCKS), jnp.float32)
    for c in range(N_CLUSTERS):
        s_row = s_ref[c:c + 1, :]
        s_col = jnp.transpose(s_row)
        m = (s_col < s_row) | ((s_col == s_row) & (q_c < p_r))
        r = jnp.sum(m.astype(jnp.float32), axis=0, keepdims=True)
        ivc = (N_SUBCLUSTERS - 1
               - (r * (1.0 / ISIZE)).astype(jnp.int32)).astype(jnp.float32)
        idxs_col = jnp.transpose(idxs_ref[c:c + 1, :])
        oh = (idxs_col == jot).astype(jnp.float32)
        acc = acc + jax.lax.dot_general(ivc, oh, (((1,), (0,)), ((), ())),
                                        preferred_element_type=jnp.float32,
                                        precision=_HI)
    iv_ref[...] = acc.astype(jnp.int32)


def _qkv_kernel(x_ref, w_ref, b_ref, o_ref):
    o_ref[...] = (jnp.dot(x_ref[...], w_ref[...],
                          preferred_element_type=jnp.float32)
                  + b_ref[...])


def _attn_kernel(q_ref, k_ref, v_ref, x_ref, clr_ref, ivr_ref, clc_ref,
                 ivc_ref, wd_ref, bd_ref, g_ref, bln_ref, wc_ref, bc_ref,
                 o_ref):
    kidx = pl.program_id(1)
    q = q_ref[...]
    k = k_ref[...]
    v = v_ref[...]
    xb = x_ref[...]
    cl_r = clr_ref[...]
    iv_r = ivr_ref[...]
    cl_c = clc_ref[...]
    iv_c = ivc_ref[...]

    member = (iv_c == kidx)
    validj = (iv_r == kidx)
    same = (cl_c == cl_r)
    addmask = jnp.where(member & same & validj, 0.0, NEG)

    ctxs = []
    for h in range(N_HEADS):
        qh = q[:, h * DH:(h + 1) * DH]
        kh = k[:, h * DH:(h + 1) * DH]
        vh = v[:, h * DH:(h + 1) * DH]
        s = jax.lax.dot_general(qh, kh, (((1,), (1,)), ((), ())),
                                preferred_element_type=jnp.float32)
        s = s * (1.0 / (DH ** 0.5)) + addmask
        m = jnp.max(s, axis=1, keepdims=True)
        p = jnp.exp(s - m)
        l = jnp.sum(p, axis=1, keepdims=True)
        ctxs.append(jnp.dot(p / l, vh, preferred_element_type=jnp.float32))
    ctx = jnp.concatenate(ctxs, axis=1)

    out = jnp.dot(ctx, wd_ref[0],
                  preferred_element_type=jnp.float32) + bd_ref[0] + xb
    mu = jnp.mean(out, axis=1, keepdims=True)
    d = out - mu
    var = jnp.mean(d * d, axis=1, keepdims=True)
    y = d / jnp.sqrt(var + 1e-12) * g_ref[0] + bln_ref[0]

    contrib = jnp.dot(y, wc_ref[0], preferred_element_type=jnp.float32)

    @pl.when(kidx == 0)
    def _():
        o_ref[...] = contrib + bc_ref[...]

    @pl.when(kidx != 0)
    def _():
        o_ref[...] = o_ref[...] + contrib


def kernel(stock_reps, market_reps, params):
    x = stock_reps
    mr = jnp.transpose(jnp.squeeze(market_reps, axis=0), (1, 0))
    sims = jnp.mean(_norm_rows(x) @ _norm_rows(mr).T, axis=1)

    cl_r, idxs = pl.pallas_call(
        _rank_kernel,
        out_shape=[
            jax.ShapeDtypeStruct((1, N_STOCKS), jnp.int32),
            jax.ShapeDtypeStruct((N_CLUSTERS, CSIZE), jnp.int32),
        ],
    )(sims.reshape(1, N_STOCKS))

    s_rows = []
    for c in range(N_CLUSTERS):
        cs = x[idxs[c]]
        centroid = jnp.mean(cs, axis=0, keepdims=True)
        s_rows.append(jnp.squeeze(_norm_rows(cs) @ _norm_rows(centroid).T))
    s8 = jnp.stack(s_rows)

    iv_r = pl.pallas_call(
        _interval_kernel,
        out_shape=jax.ShapeDtypeStruct((1, N_STOCKS), jnp.int32),
    )(s8, idxs)

    attn = params['attn']
    wcat = jnp.concatenate(
        [attn[k][nm] for k in range(N_SUBCLUSTERS)
         for nm in ('Wq', 'Wk', 'Wv')], axis=1)
    bcat = jnp.concatenate(
        [attn[k][nm] for k in range(N_SUBCLUSTERS)
         for nm in ('bq', 'bk', 'bv')], axis=0)[None, :]

    qkv = pl.pallas_call(
        _qkv_kernel,
        grid=(N_STOCKS // QB,),
        in_specs=[
            pl.BlockSpec((QB, HIDDEN), lambda i: (i, 0)),
            pl.BlockSpec((HIDDEN, 3 * N_SUBCLUSTERS * HIDDEN),
                         lambda i: (0, 0)),
            pl.BlockSpec((1, 3 * N_SUBCLUSTERS * HIDDEN), lambda i: (0, 0)),
        ],
        out_specs=pl.BlockSpec((QB, 3 * N_SUBCLUSTERS * HIDDEN),
                               lambda i: (i, 0)),
        out_shape=jax.ShapeDtypeStruct(
            (N_STOCKS, 3 * N_SUBCLUSTERS * HIDDEN), jnp.float32),
    )(x, wcat, bcat)

    wd = jnp.stack([attn[k]['Wd'] for k in range(N_SUBCLUSTERS)])
    bd = jnp.stack([attn[k]['bd'] for k in range(N_SUBCLUSTERS)])[:, None, :]
    g = jnp.stack([attn[k]['g'] for k in range(N_SUBCLUSTERS)])[:, None, :]
    bln = jnp.stack([attn[k]['b_ln']
                     for k in range(N_SUBCLUSTERS)])[:, None, :]
    wc = params['Wc'].reshape(N_SUBCLUSTERS, HIDDEN, HIDDEN)
    bc = params['bc'][None, :]
    cl_c = cl_r.reshape(N_STOCKS, 1)
    iv_c = iv_r.reshape(N_STOCKS, 1)

    out = pl.pallas_call(
        _attn_kernel,
        grid=(N_STOCKS // QB, N_SUBCLUSTERS),
        in_specs=[
            pl.BlockSpec((QB, HIDDEN), lambda i, k: (i, 3 * k)),
            pl.BlockSpec((N_STOCKS, HIDDEN), lambda i, k: (0, 3 * k + 1)),
            pl.BlockSpec((N_STOCKS, HIDDEN), lambda i, k: (0, 3 * k + 2)),
            pl.BlockSpec((QB, HIDDEN), lambda i, k: (i, 0)),
            pl.BlockSpec((1, N_STOCKS), lambda i, k: (0, 0)),
            pl.BlockSpec((1, N_STOCKS), lambda i, k: (0, 0)),
            pl.BlockSpec((QB, 1), lambda i, k: (i, 0)),
            pl.BlockSpec((QB, 1), lambda i, k: (i, 0)),
            pl.BlockSpec((1, HIDDEN, HIDDEN), lambda i, k: (k, 0, 0)),
            pl.BlockSpec((1, 1, HIDDEN), lambda i, k: (k, 0, 0)),
            pl.BlockSpec((1, 1, HIDDEN), lambda i, k: (k, 0, 0)),
            pl.BlockSpec((1, 1, HIDDEN), lambda i, k: (k, 0, 0)),
            pl.BlockSpec((1, HIDDEN, HIDDEN), lambda i, k: (k, 0, 0)),
            pl.BlockSpec((1, HIDDEN), lambda i, k: (0, 0)),
        ],
        out_specs=pl.BlockSpec((QB, HIDDEN), lambda i, k: (i, 0)),
        out_shape=jax.ShapeDtypeStruct((N_STOCKS, HIDDEN), jnp.float32),
    )(qkv, qkv, qkv, x, cl_r, iv_r, cl_c, iv_c, wd, bd, g, bln, wc, bc)

    return out, cl_r.reshape(-1), sims


# speedup vs baseline: 2.9542x; 1.3746x over previous
import jax
import jax.numpy as jnp
from jax.experimental import pallas as pl

N_STOCKS = 2048
N_MARKET = 32
HIDDEN = 256
N_CLUSTERS = 8
N_SUBCLUSTERS = 4
N_HEADS = 4
DH = HIDDEN // N_HEADS
CSIZE = N_STOCKS // N_CLUSTERS
ISIZE = CSIZE // N_SUBCLUSTERS
QB = CSIZE
W3 = 3 * N_SUBCLUSTERS * HIDDEN
NEG = -1000000000.0
_HI = jax.lax.Precision.HIGHEST


def _norm_rows(v, eps=1e-12):
    n = jnp.sqrt(jnp.sum(v * v, axis=-1, keepdims=True))
    return v / jnp.maximum(n, eps)


def _rank_kernel(sims_ref, cl_ref, idxs_ref):
    sims_r = sims_ref[...]
    sims_c = jnp.transpose(sims_r)
    idx_c = jax.lax.broadcasted_iota(jnp.int32, (N_STOCKS, 1), 0)
    idx_r = jax.lax.broadcasted_iota(jnp.int32, (1, N_STOCKS), 1)

    lt = (sims_r < sims_c) | ((sims_r == sims_c) & (idx_r < idx_c))
    rank = jnp.sum(lt.astype(jnp.float32), axis=1, keepdims=True)
    cl_c = (N_CLUSTERS - 1
            - (rank * (1.0 / CSIZE)).astype(jnp.int32)).astype(jnp.int32)
    cl_r = jnp.transpose(cl_c)

    same = (cl_r == cl_c)
    pos = jnp.sum((same & (idx_r < idx_c)).astype(jnp.float32), axis=1,
                  keepdims=True)
    piota = jax.lax.broadcasted_iota(jnp.int32, (1, CSIZE), 1)
    pos_i = pos.astype(jnp.int32)
    jf_r = idx_r.astype(jnp.float32)

    cl_ref[...] = cl_r
    for c in range(N_CLUSTERS):
        memb_c = (cl_c == c)
        ph = ((pos_i == piota) & memb_c).astype(jnp.float32)
        row = jax.lax.dot_general(jf_r, ph, (((1,), (0,)), ((), ())),
                                  preferred_element_type=jnp.float32,
                                  precision=_HI)
        idxs_ref[c, :] = row.astype(jnp.int32)[0]


def _interval_kernel(s_ref, iv_ref):
    q_c = jax.lax.broadcasted_iota(jnp.int32, (CSIZE, 1), 0)
    p_r = jax.lax.broadcasted_iota(jnp.int32, (1, CSIZE), 1)
    for c in range(N_CLUSTERS):
        s_row = s_ref[c:c + 1, :]
        s_col = jnp.transpose(s_row)
        m = (s_col < s_row) | ((s_col == s_row) & (q_c < p_r))
        r = jnp.sum(m.astype(jnp.float32), axis=0, keepdims=True)
        iv_ref[c, :] = (N_SUBCLUSTERS - 1
                        - (r * (1.0 / ISIZE)).astype(jnp.int32))[0]


def _qkv_kernel(x_ref, w_ref, b_ref, o_ref, vs_ref):
    i = pl.program_id(0)
    o = (jnp.dot(x_ref[...], w_ref[...],
                 preferred_element_type=jnp.float32) + b_ref[...])
    o_ref[...] = o
    cs = jnp.sum(o, axis=0, keepdims=True)

    @pl.when(i == 0)
    def _():
        vs_ref[...] = cs

    @pl.when(i != 0)
    def _():
        vs_ref[...] = vs_ref[...] + cs


def _attn_kernel(q_ref, k_ref, v_ref, vs_ref, x_ref, ivr_ref, ivc_ref,
                 wd_ref, bd_ref, g_ref, bln_ref, wc_ref, bc_ref, o_ref):
    kidx = pl.program_id(1)
    q = q_ref[...]
    k = k_ref[...]
    v = v_ref[...]
    xb = x_ref[...]
    iv_r = ivr_ref[0]
    iv_c = ivc_ref[0]

    member_q = (iv_c == kidx)
    member_k = (iv_r == kidx)
    addmask = jnp.where(member_q & member_k, 0.0, NEG)

    meanv = vs_ref[0] * (1.0 / N_STOCKS)

    ctxs = []
    for h in range(N_HEADS):
        qh = q[:, h * DH:(h + 1) * DH]
        kh = k[:, h * DH:(h + 1) * DH]
        vh = v[:, h * DH:(h + 1) * DH]
        s = jax.lax.dot_general(qh, kh, (((1,), (1,)), ((), ())),
                                preferred_element_type=jnp.float32)
        s = s * (1.0 / (DH ** 0.5)) + addmask
        m = jnp.max(s, axis=1, keepdims=True)
        p = jnp.exp(s - m)
        l = jnp.sum(p, axis=1, keepdims=True)
        ctxs.append(jnp.dot(p / l, vh, preferred_element_type=jnp.float32))
    ctx = jnp.concatenate(ctxs, axis=1)
    ctx = jnp.where(member_q, ctx, meanv)

    out = jnp.dot(ctx, wd_ref[0],
                  preferred_element_type=jnp.float32) + bd_ref[0] + xb
    mu = jnp.mean(out, axis=1, keepdims=True)
    d = out - mu
    var = jnp.mean(d * d, axis=1, keepdims=True)
    y = d / jnp.sqrt(var + 1e-12) * g_ref[0] + bln_ref[0]

    contrib = jnp.dot(y, wc_ref[0], preferred_element_type=jnp.float32)

    @pl.when(kidx == 0)
    def _():
        o_ref[...] = contrib + bc_ref[...]

    @pl.when(kidx != 0)
    def _():
        o_ref[...] = o_ref[...] + contrib


def kernel(stock_reps, market_reps, params):
    x = stock_reps
    mr = jnp.transpose(jnp.squeeze(market_reps, axis=0), (1, 0))
    sims = jnp.mean(_norm_rows(x) @ _norm_rows(mr).T, axis=1)

    cl_r, idxs = pl.pallas_call(
        _rank_kernel,
        out_shape=[
            jax.ShapeDtypeStruct((1, N_STOCKS), jnp.int32),
            jax.ShapeDtypeStruct((N_CLUSTERS, CSIZE), jnp.int32),
        ],
    )(sims.reshape(1, N_STOCKS))

    s_rows = []
    for c in range(N_CLUSTERS):
        cs = x[idxs[c]]
        centroid = jnp.mean(cs, axis=0, keepdims=True)
        s_rows.append(jnp.squeeze(_norm_rows(cs) @ _norm_rows(centroid).T))
    s8 = jnp.stack(s_rows)

    iv8 = pl.pallas_call(
        _interval_kernel,
        out_shape=jax.ShapeDtypeStruct((N_CLUSTERS, CSIZE), jnp.int32),
    )(s8)

    perm = idxs.reshape(N_STOCKS)
    x_perm = x[perm]

    attn = params['attn']
    wcat = jnp.concatenate(
        [attn[k][nm] for k in range(N_SUBCLUSTERS)
         for nm in ('Wq', 'Wk', 'Wv')], axis=1)
    bcat = jnp.concatenate(
        [attn[k][nm] for k in range(N_SUBCLUSTERS)
         for nm in ('bq', 'bk', 'bv')], axis=0)[None, :]

    qkv, vsum = pl.pallas_call(
        _qkv_kernel,
        grid=(N_STOCKS // QB,),
        in_specs=[
            pl.BlockSpec((QB, HIDDEN), lambda i: (i, 0)),
            pl.BlockSpec((HIDDEN, W3), lambda i: (0, 0)),
            pl.BlockSpec((1, W3), lambda i: (0, 0)),
        ],
        out_specs=[
            pl.BlockSpec((QB, W3), lambda i: (i, 0)),
            pl.BlockSpec((1, W3), lambda i: (0, 0)),
        ],
        out_shape=[
            jax.ShapeDtypeStruct((N_STOCKS, W3), jnp.float32),
            jax.ShapeDtypeStruct((1, W3), jnp.float32),
        ],
    )(x_perm, wcat, bcat)

    wd = jnp.stack([attn[k]['Wd'] for k in range(N_SUBCLUSTERS)])
    bd = jnp.stack([attn[k]['bd'] for k in range(N_SUBCLUSTERS)])[:, None, :]
    g = jnp.stack([attn[k]['g'] for k in range(N_SUBCLUSTERS)])[:, None, :]
    bln = jnp.stack([attn[k]['b_ln']
                     for k in range(N_SUBCLUSTERS)])[:, None, :]
    wc = params['Wc'].reshape(N_SUBCLUSTERS, HIDDEN, HIDDEN)
    bc = params['bc'][None, :]
    iv_row = iv8.reshape(N_CLUSTERS, 1, CSIZE)
    iv_col = iv8.reshape(N_CLUSTERS, CSIZE, 1)

    out_perm = pl.pallas_call(
        _attn_kernel,
        grid=(N_CLUSTERS, N_SUBCLUSTERS),
        in_specs=[
            pl.BlockSpec((QB, HIDDEN), lambda i, k: (i, 3 * k)),
            pl.BlockSpec((QB, HIDDEN), lambda i, k: (i, 3 * k + 1)),
            pl.BlockSpec((QB, HIDDEN), lambda i, k: (i, 3 * k + 2)),
            pl.BlockSpec((1, HIDDEN), lambda i, k: (0, 3 * k + 2)),
            pl.BlockSpec((QB, HIDDEN), lambda i, k: (i, 0)),
            pl.BlockSpec((1, 1, CSIZE), lambda i, k: (i, 0, 0)),
            pl.BlockSpec((1, CSIZE, 1), lambda i, k: (i, 0, 0)),
            pl.BlockSpec((1, HIDDEN, HIDDEN), lambda i, k: (k, 0, 0)),
            pl.BlockSpec((1, 1, HIDDEN), lambda i, k: (k, 0, 0)),
            pl.BlockSpec((1, 1, HIDDEN), lambda i, k: (k, 0, 0)),
            pl.BlockSpec((1, 1, HIDDEN), lambda i, k: (k, 0, 0)),
            pl.BlockSpec((1, HIDDEN, HIDDEN), lambda i, k: (k, 0, 0)),
            pl.BlockSpec((1, HIDDEN), lambda i, k: (0, 0)),
        ],
        out_specs=pl.BlockSpec((QB, HIDDEN), lambda i, k: (i, 0)),
        out_shape=jax.ShapeDtypeStruct((N_STOCKS, HIDDEN), jnp.float32),
    )(qkv, qkv, qkv, vsum, x_perm, iv_row, iv_col, wd, bd, g, bln, wc, bc)

    reps = jnp.zeros((N_STOCKS, HIDDEN), jnp.float32).at[perm].set(out_perm)
    return reps, cl_r.reshape(-1), sims


# speedup vs baseline: 3.4646x; 1.1728x over previous
import jax
import jax.numpy as jnp
from jax.experimental import pallas as pl

N_STOCKS = 2048
N_MARKET = 32
HIDDEN = 256
N_CLUSTERS = 8
N_SUBCLUSTERS = 4
N_HEADS = 4
DH = HIDDEN // N_HEADS
CSIZE = N_STOCKS // N_CLUSTERS
ISIZE = CSIZE // N_SUBCLUSTERS
QB = CSIZE
W3 = 3 * N_SUBCLUSTERS * HIDDEN
NEG = -1000000000.0
_HI = jax.lax.Precision.HIGHEST


def _norm_rows(v, eps=1e-12):
    n = jnp.sqrt(jnp.sum(v * v, axis=-1, keepdims=True))
    return v / jnp.maximum(n, eps)


def _rank_kernel(sims_ref, cl_ref, idxs_ref):
    sims_r = sims_ref[...]
    sims_c = jnp.transpose(sims_r)
    idx_c = jax.lax.broadcasted_iota(jnp.int32, (N_STOCKS, 1), 0)
    idx_r = jax.lax.broadcasted_iota(jnp.int32, (1, N_STOCKS), 1)

    lt = (sims_r < sims_c) | ((sims_r == sims_c) & (idx_r < idx_c))
    rank = jnp.sum(lt.astype(jnp.float32), axis=1, keepdims=True)
    cl_c = (N_CLUSTERS - 1
            - (rank * (1.0 / CSIZE)).astype(jnp.int32)).astype(jnp.int32)
    cl_r = jnp.transpose(cl_c)

    same = (cl_r == cl_c)
    pos = jnp.sum((same & (idx_r < idx_c)).astype(jnp.float32), axis=1,
                  keepdims=True)
    piota = jax.lax.broadcasted_iota(jnp.int32, (1, CSIZE), 1)
    pos_i = pos.astype(jnp.int32)
    jf_r = idx_r.astype(jnp.float32)

    cl_ref[...] = cl_r
    for c in range(N_CLUSTERS):
        memb_c = (cl_c == c)
        ph = ((pos_i == piota) & memb_c).astype(jnp.float32)
        row = jax.lax.dot_general(jf_r, ph, (((1,), (0,)), ((), ())),
                                  preferred_element_type=jnp.float32,
                                  precision=_HI)
        idxs_ref[c, :] = row.astype(jnp.int32)[0]


def _interval_kernel(s_ref, iv_ref):
    q_c = jax.lax.broadcasted_iota(jnp.int32, (CSIZE, 1), 0)
    p_r = jax.lax.broadcasted_iota(jnp.int32, (1, CSIZE), 1)
    for c in range(N_CLUSTERS):
        s_row = s_ref[c:c + 1, :]
        s_col = jnp.transpose(s_row)
        m = (s_col < s_row) | ((s_col == s_row) & (q_c < p_r))
        r = jnp.sum(m.astype(jnp.float32), axis=0, keepdims=True)
        iv_ref[c, :] = (N_SUBCLUSTERS - 1
                        - (r * (1.0 / ISIZE)).astype(jnp.int32))[0]


def _qkv_kernel(x_ref, w_ref, b_ref, o_ref, vs_ref):
    i = pl.program_id(0)
    o = (jnp.dot(x_ref[...], w_ref[...],
                 preferred_element_type=jnp.float32) + b_ref[...])
    o_ref[...] = o
    cs = jnp.sum(o, axis=0, keepdims=True)

    @pl.when(i == 0)
    def _():
        vs_ref[...] = cs

    @pl.when(i != 0)
    def _():
        vs_ref[...] = vs_ref[...] + cs


def _attn_kernel(q_ref, k_ref, v_ref, vs_ref, x_ref, ivr_ref, ivc_ref,
                 wd_ref, bd_ref, g_ref, bln_ref, wc_ref, bc_ref, o_ref):
    kidx = pl.program_id(1)
    q = q_ref[...]
    k = k_ref[...]
    v = v_ref[...]
    xb = x_ref[...]
    iv_r = ivr_ref[0]
    iv_c = ivc_ref[0]

    member_q = (iv_c == kidx)
    member_k = (iv_r == kidx)
    addmask = jnp.where(member_q & member_k, 0.0, NEG)

    meanv = vs_ref[0] * (1.0 / N_STOCKS)

    ctxs = []
    for h in range(N_HEADS):
        qh = q[:, h * DH:(h + 1) * DH]
        kh = k[:, h * DH:(h + 1) * DH]
        vh = v[:, h * DH:(h + 1) * DH]
        s = jax.lax.dot_general(qh, kh, (((1,), (1,)), ((), ())),
                                preferred_element_type=jnp.float32)
        s = s * (1.0 / (DH ** 0.5)) + addmask
        m = jnp.max(s, axis=1, keepdims=True)
        p = jnp.exp(s - m)
        l = jnp.sum(p, axis=1, keepdims=True)
        ctxs.append(jnp.dot(p / l, vh, preferred_element_type=jnp.float32))
    ctx = jnp.concatenate(ctxs, axis=1)
    ctx = jnp.where(member_q, ctx, meanv)

    out = jnp.dot(ctx, wd_ref[0],
                  preferred_element_type=jnp.float32) + bd_ref[0] + xb
    mu = jnp.mean(out, axis=1, keepdims=True)
    d = out - mu
    var = jnp.mean(d * d, axis=1, keepdims=True)
    y = d / jnp.sqrt(var + 1e-12) * g_ref[0] + bln_ref[0]

    contrib = jnp.dot(y, wc_ref[0], preferred_element_type=jnp.float32)

    @pl.when(kidx == 0)
    def _():
        o_ref[...] = contrib + bc_ref[...]

    @pl.when(kidx != 0)
    def _():
        o_ref[...] = o_ref[...] + contrib


def kernel(stock_reps, market_reps, params):
    x = stock_reps
    mr = jnp.transpose(jnp.squeeze(market_reps, axis=0), (1, 0))
    sims = jnp.mean(_norm_rows(x) @ _norm_rows(mr).T, axis=1)

    cl_r, idxs = pl.pallas_call(
        _rank_kernel,
        out_shape=[
            jax.ShapeDtypeStruct((1, N_STOCKS), jnp.int32),
            jax.ShapeDtypeStruct((N_CLUSTERS, CSIZE), jnp.int32),
        ],
    )(sims.reshape(1, N_STOCKS))

    perm = idxs.reshape(N_STOCKS)
    x_perm = x[perm]

    s_rows = []
    for c in range(N_CLUSTERS):
        cs = x_perm[c * CSIZE:(c + 1) * CSIZE]
        centroid = jnp.mean(cs, axis=0, keepdims=True)
        s_rows.append(jnp.squeeze(_norm_rows(cs) @ _norm_rows(centroid).T))
    s8 = jnp.stack(s_rows)

    iv8 = pl.pallas_call(
        _interval_kernel,
        out_shape=jax.ShapeDtypeStruct((N_CLUSTERS, CSIZE), jnp.int32),
    )(s8)

    attn = params['attn']
    wcat = jnp.concatenate(
        [attn[k][nm] for k in range(N_SUBCLUSTERS)
         for nm in ('Wq', 'Wk', 'Wv')], axis=1)
    bcat = jnp.concatenate(
        [attn[k][nm] for k in range(N_SUBCLUSTERS)
         for nm in ('bq', 'bk', 'bv')], axis=0)[None, :]

    qkv, vsum = pl.pallas_call(
        _qkv_kernel,
        grid=(N_STOCKS // QB,),
        in_specs=[
            pl.BlockSpec((QB, HIDDEN), lambda i: (i, 0)),
            pl.BlockSpec((HIDDEN, W3), lambda i: (0, 0)),
            pl.BlockSpec((1, W3), lambda i: (0, 0)),
        ],
        out_specs=[
            pl.BlockSpec((QB, W3), lambda i: (i, 0)),
            pl.BlockSpec((1, W3), lambda i: (0, 0)),
        ],
        out_shape=[
            jax.ShapeDtypeStruct((N_STOCKS, W3), jnp.float32),
            jax.ShapeDtypeStruct((1, W3), jnp.float32),
        ],
    )(x_perm, wcat, bcat)

    wd = jnp.stack([attn[k]['Wd'] for k in range(N_SUBCLUSTERS)])
    bd = jnp.stack([attn[k]['bd'] for k in range(N_SUBCLUSTERS)])[:, None, :]
    g = jnp.stack([attn[k]['g'] for k in range(N_SUBCLUSTERS)])[:, None, :]
    bln = jnp.stack([attn[k]['b_ln']
                     for k in range(N_SUBCLUSTERS)])[:, None, :]
    wc = params['Wc'].reshape(N_SUBCLUSTERS, HIDDEN, HIDDEN)
    bc = params['bc'][None, :]
    iv_row = iv8.reshape(N_CLUSTERS, 1, CSIZE)
    iv_col = iv8.reshape(N_CLUSTERS, CSIZE, 1)

    out_perm = pl.pallas_call(
        _attn_kernel,
        grid=(N_CLUSTERS, N_SUBCLUSTERS),
        in_specs=[
            pl.BlockSpec((QB, HIDDEN), lambda i, k: (i, 3 * k)),
            pl.BlockSpec((QB, HIDDEN), lambda i, k: (i, 3 * k + 1)),
            pl.BlockSpec((QB, HIDDEN), lambda i, k: (i, 3 * k + 2)),
            pl.BlockSpec((1, HIDDEN), lambda i, k: (0, 3 * k + 2)),
            pl.BlockSpec((QB, HIDDEN), lambda i, k: (i, 0)),
            pl.BlockSpec((1, 1, CSIZE), lambda i, k: (i, 0, 0)),
            pl.BlockSpec((1, CSIZE, 1), lambda i, k: (i, 0, 0)),
            pl.BlockSpec((1, HIDDEN, HIDDEN), lambda i, k: (k, 0, 0)),
            pl.BlockSpec((1, 1, HIDDEN), lambda i, k: (k, 0, 0)),
            pl.BlockSpec((1, 1, HIDDEN), lambda i, k: (k, 0, 0)),
            pl.BlockSpec((1, 1, HIDDEN), lambda i, k: (k, 0, 0)),
            pl.BlockSpec((1, HIDDEN, HIDDEN), lambda i, k: (k, 0, 0)),
            pl.BlockSpec((1, HIDDEN), lambda i, k: (0, 0)),
        ],
        out_specs=pl.BlockSpec((QB, HIDDEN), lambda i, k: (i, 0)),
        out_shape=jax.ShapeDtypeStruct((N_STOCKS, HIDDEN), jnp.float32),
    )(qkv, qkv, qkv, vsum, x_perm, iv_row, iv_col, wd, bd, g, bln, wc, bc)

    reps = jnp.zeros((N_STOCKS, HIDDEN), jnp.float32).at[perm].set(out_perm)
    return reps, cl_r.reshape(-1), sims


# speedup vs baseline: 3.5036x; 1.0113x over previous
import functools

import jax
import jax.numpy as jnp
from jax import lax
from jax.experimental import pallas as pl
from jax.experimental.pallas import tpu as pltpu
from jax.experimental.pallas import tpu_sc as plsc

N_STOCKS = 2048
N_MARKET = 32
HIDDEN = 256
N_CLUSTERS = 8
N_SUBCLUSTERS = 4
N_HEADS = 4
DH = HIDDEN // N_HEADS
CSIZE = N_STOCKS // N_CLUSTERS
ISIZE = CSIZE // N_SUBCLUSTERS
QB = CSIZE
W3 = 3 * N_SUBCLUSTERS * HIDDEN
NEG = -1000000000.0
_HI = jax.lax.Precision.HIGHEST


def _norm_rows(v, eps=1e-12):
    n = jnp.sqrt(jnp.sum(v * v, axis=-1, keepdims=True))
    return v / jnp.maximum(n, eps)


def _rank_kernel(sims_ref, cl_ref, idxs_ref):
    sims_r = sims_ref[...]
    sims_c = jnp.transpose(sims_r)
    idx_c = jax.lax.broadcasted_iota(jnp.int32, (N_STOCKS, 1), 0)
    idx_r = jax.lax.broadcasted_iota(jnp.int32, (1, N_STOCKS), 1)

    lt = (sims_r < sims_c) | ((sims_r == sims_c) & (idx_r < idx_c))
    rank = jnp.sum(lt.astype(jnp.float32), axis=1, keepdims=True)
    cl_c = (N_CLUSTERS - 1
            - (rank * (1.0 / CSIZE)).astype(jnp.int32)).astype(jnp.int32)
    cl_r = jnp.transpose(cl_c)

    same = (cl_r == cl_c)
    pos = jnp.sum((same & (idx_r < idx_c)).astype(jnp.float32), axis=1,
                  keepdims=True)
    piota = jax.lax.broadcasted_iota(jnp.int32, (1, CSIZE), 1)
    pos_i = pos.astype(jnp.int32)
    jf_r = idx_r.astype(jnp.float32)

    cl_ref[...] = cl_r
    for c in range(N_CLUSTERS):
        memb_c = (cl_c == c)
        ph = ((pos_i == piota) & memb_c).astype(jnp.float32)
        row = jax.lax.dot_general(jf_r, ph, (((1,), (0,)), ((), ())),
                                  preferred_element_type=jnp.float32,
                                  precision=_HI)
        idxs_ref[c, :] = row.astype(jnp.int32)[0]


def _interval_sc_kernel(s_hbm, iv_hbm, s_v, iv_v):
    wid = lax.axis_index("s") * 2 + lax.axis_index("c")
    c = wid // N_SUBCLUSTERS
    q = wid % N_SUBCLUSTERS
    pltpu.sync_copy(s_hbm.at[c], s_v)
    for t in range(ISIZE // 16):
        base = q * ISIZE + t * 16
        s_mine = s_v[pl.ds(base, 16)]
        posv = lax.iota(jnp.int32, 16) + base

        def body(jb, cnt):
            sb = s_v[pl.ds(jb * 16, 16)]
            for l in range(16):
                j = jb * 16 + l
                sj = sb[l]
                beats = (sj < s_mine) | ((sj == s_mine) & (j < posv))
                cnt = cnt + jnp.where(beats, 1, 0)
            return cnt

        cnt = lax.fori_loop(0, CSIZE // 16, body,
                            jnp.zeros((16,), jnp.int32))
        iv_v[pl.ds(t * 16, 16)] = (N_SUBCLUSTERS - 1
                                   - jax.lax.shift_right_logical(cnt, 6))
    pltpu.sync_copy(iv_v, iv_hbm.at[c, pl.ds(q * ISIZE, ISIZE)])


_interval_sc = functools.partial(
    pl.kernel,
    mesh=plsc.VectorSubcoreMesh(core_axis_name="c", subcore_axis_name="s"),
    out_type=jax.ShapeDtypeStruct((N_CLUSTERS, CSIZE), jnp.int32),
    scratch_types=[
        pltpu.VMEM((CSIZE,), jnp.float32),
        pltpu.VMEM((ISIZE,), jnp.int32),
    ],
)(_interval_sc_kernel)


def _qkv_kernel(x_ref, w_ref, b_ref, o_ref, vs_ref):
    i = pl.program_id(0)
    o = (jnp.dot(x_ref[...], w_ref[...],
                 preferred_element_type=jnp.float32) + b_ref[...])
    o_ref[...] = o
    cs = jnp.sum(o, axis=0, keepdims=True)

    @pl.when(i == 0)
    def _():
        vs_ref[...] = cs

    @pl.when(i != 0)
    def _():
        vs_ref[...] = vs_ref[...] + cs


def _attn_kernel(q_ref, k_ref, v_ref, vs_ref, x_ref, ivr_ref, ivc_ref,
                 wd_ref, bd_ref, g_ref, bln_ref, wc_ref, bc_ref, o_ref):
    kidx = pl.program_id(1)
    q = q_ref[...]
    k = k_ref[...]
    v = v_ref[...]
    xb = x_ref[...]
    iv_r = ivr_ref[0]
    iv_c = ivc_ref[0]

    member_q = (iv_c == kidx)
    member_k = (iv_r == kidx)
    addmask = jnp.where(member_q & member_k, 0.0, NEG)

    meanv = vs_ref[0] * (1.0 / N_STOCKS)

    ctxs = []
    for h in range(N_HEADS):
        qh = q[:, h * DH:(h + 1) * DH]
        kh = k[:, h * DH:(h + 1) * DH]
        vh = v[:, h * DH:(h + 1) * DH]
        s = jax.lax.dot_general(qh, kh, (((1,), (1,)), ((), ())),
                                preferred_element_type=jnp.float32)
        s = s * (1.0 / (DH ** 0.5)) + addmask
        m = jnp.max(s, axis=1, keepdims=True)
        p = jnp.exp(s - m)
        l = jnp.sum(p, axis=1, keepdims=True)
        ctxs.append(jnp.dot(p / l, vh, preferred_element_type=jnp.float32))
    ctx = jnp.concatenate(ctxs, axis=1)
    ctx = jnp.where(member_q, ctx, meanv)

    out = jnp.dot(ctx, wd_ref[0],
                  preferred_element_type=jnp.float32) + bd_ref[0] + xb
    mu = jnp.mean(out, axis=1, keepdims=True)
    d = out - mu
    var = jnp.mean(d * d, axis=1, keepdims=True)
    y = d / jnp.sqrt(var + 1e-12) * g_ref[0] + bln_ref[0]

    contrib = jnp.dot(y, wc_ref[0], preferred_element_type=jnp.float32)

    @pl.when(kidx == 0)
    def _():
        o_ref[...] = contrib + bc_ref[...]

    @pl.when(kidx != 0)
    def _():
        o_ref[...] = o_ref[...] + contrib


def kernel(stock_reps, market_reps, params):
    x = stock_reps
    mr = jnp.transpose(jnp.squeeze(market_reps, axis=0), (1, 0))
    sims = jnp.mean(_norm_rows(x) @ _norm_rows(mr).T, axis=1)

    cl_r, idxs = pl.pallas_call(
        _rank_kernel,
        out_shape=[
            jax.ShapeDtypeStruct((1, N_STOCKS), jnp.int32),
            jax.ShapeDtypeStruct((N_CLUSTERS, CSIZE), jnp.int32),
        ],
    )(sims.reshape(1, N_STOCKS))

    perm = idxs.reshape(N_STOCKS)
    x_perm = x[perm]

    s_rows = []
    for c in range(N_CLUSTERS):
        cs = x_perm[c * CSIZE:(c + 1) * CSIZE]
        centroid = jnp.mean(cs, axis=0, keepdims=True)
        s_rows.append(jnp.squeeze(_norm_rows(cs) @ _norm_rows(centroid).T))
    s8 = jnp.stack(s_rows)

    iv8 = _interval_sc(s8)

    attn = params['attn']
    wcat = jnp.concatenate(
        [attn[k][nm] for k in range(N_SUBCLUSTERS)
         for nm in ('Wq', 'Wk', 'Wv')], axis=1)
    bcat = jnp.concatenate(
        [attn[k][nm] for k in range(N_SUBCLUSTERS)
         for nm in ('bq', 'bk', 'bv')], axis=0)[None, :]

    qkv, vsum = pl.pallas_call(
        _qkv_kernel,
        grid=(N_STOCKS // QB,),
        in_specs=[
            pl.BlockSpec((QB, HIDDEN), lambda i: (i, 0)),
            pl.BlockSpec((HIDDEN, W3), lambda i: (0, 0)),
            pl.BlockSpec((1, W3), lambda i: (0, 0)),
        ],
        out_specs=[
            pl.BlockSpec((QB, W3), lambda i: (i, 0)),
            pl.BlockSpec((1, W3), lambda i: (0, 0)),
        ],
        out_shape=[
            jax.ShapeDtypeStruct((N_STOCKS, W3), jnp.float32),
            jax.ShapeDtypeStruct((1, W3), jnp.float32),
        ],
    )(x_perm, wcat, bcat)

    wd = jnp.stack([attn[k]['Wd'] for k in range(N_SUBCLUSTERS)])
    bd = jnp.stack([attn[k]['bd'] for k in range(N_SUBCLUSTERS)])[:, None, :]
    g = jnp.stack([attn[k]['g'] for k in range(N_SUBCLUSTERS)])[:, None, :]
    bln = jnp.stack([attn[k]['b_ln']
                     for k in range(N_SUBCLUSTERS)])[:, None, :]
    wc = params['Wc'].reshape(N_SUBCLUSTERS, HIDDEN, HIDDEN)
    bc = params['bc'][None, :]
    iv_row = iv8.reshape(N_CLUSTERS, 1, CSIZE)
    iv_col = iv8.reshape(N_CLUSTERS, CSIZE, 1)

    out_perm = pl.pallas_call(
        _attn_kernel,
        grid=(N_CLUSTERS, N_SUBCLUSTERS),
        in_specs=[
            pl.BlockSpec((QB, HIDDEN), lambda i, k: (i, 3 * k)),
            pl.BlockSpec((QB, HIDDEN), lambda i, k: (i, 3 * k + 1)),
            pl.BlockSpec((QB, HIDDEN), lambda i, k: (i, 3 * k + 2)),
            pl.BlockSpec((1, HIDDEN), lambda i, k: (0, 3 * k + 2)),
            pl.BlockSpec((QB, HIDDEN), lambda i, k: (i, 0)),
            pl.BlockSpec((1, 1, CSIZE), lambda i, k: (i, 0, 0)),
            pl.BlockSpec((1, CSIZE, 1), lambda i, k: (i, 0, 0)),
            pl.BlockSpec((1, HIDDEN, HIDDEN), lambda i, k: (k, 0, 0)),
            pl.BlockSpec((1, 1, HIDDEN), lambda i, k: (k, 0, 0)),
            pl.BlockSpec((1, 1, HIDDEN), lambda i, k: (k, 0, 0)),
            pl.BlockSpec((1, 1, HIDDEN), lambda i, k: (k, 0, 0)),
            pl.BlockSpec((1, HIDDEN, HIDDEN), lambda i, k: (k, 0, 0)),
            pl.BlockSpec((1, HIDDEN), lambda i, k: (0, 0)),
        ],
        out_specs=pl.BlockSpec((QB, HIDDEN), lambda i, k: (i, 0)),
        out_shape=jax.ShapeDtypeStruct((N_STOCKS, HIDDEN), jnp.float32),
    )(qkv, qkv, qkv, vsum, x_perm, iv_row, iv_col, wd, bd, g, bln, wc, bc)

    reps = jnp.zeros((N_STOCKS, HIDDEN), jnp.float32).at[perm].set(out_perm)
    return reps, cl_r.reshape(-1), sims
